# Initial kernel scaffold; baseline (speedup 1.0000x reference)
import jax
import jax.numpy as jnp
from jax.experimental import pallas as pl


def kernel(x, emb, w_lin, b_lin, W1, b1, g1, be1, W2, b2, g2, be2, W3, b3):
    raise NotImplementedError("write your pallas kernel here")


# speedup vs baseline: 29.2063x; 29.2063x over previous
import functools

import jax
import jax.numpy as jnp
from jax import lax
from jax.experimental import pallas as pl
from jax.experimental.pallas import tpu as pltpu
from jax.experimental.pallas import tpu_sc as plsc

B = 4096
F = 26
V = 1000
D = 16
H1, H2 = 64, 32
EPS = 1e-5

NC = 2
NS = 16
NW = NC * NS
E_PER_W = (B * F) // NW
CH = E_PER_W // 128
LANE_STEPS = E_PER_W // 16


def _sc_gather(idx2d_hbm, xfm_hbm, wtab_hbm, emb_hbm,
               rows_out, wg_out,
               idx_v, rows_v, xv, wgv, wtab, sem):
    wid = lax.axis_index("s") * NC + lax.axis_index("c")
    base = wid * E_PER_W

    pltpu.sync_copy(idx2d_hbm.at[wid], idx_v)
    pltpu.sync_copy(xfm_hbm.at[pl.ds(base, E_PER_W)], xv)
    pltpu.sync_copy(wtab_hbm.at[pl.ds(0, V)], wtab)

    def fire(j, c):
        pltpu.async_copy(emb_hbm.at[idx_v.at[j]], rows_v.at[j], sem)
        return c
    lax.fori_loop(0, CH, fire, 0)

    def wbody(i, c):
        idx = xv[pl.ds(i * 16, 16)]
        wgv[pl.ds(i * 16, 16)] = plsc.load_gather(wtab, [idx])
        return c
    lax.fori_loop(0, LANE_STEPS, wbody, 0)
    pltpu.sync_copy(wgv, wg_out.at[pl.ds(base, E_PER_W)])

    def drain(j, c):
        pltpu.make_async_copy(emb_hbm.at[idx_v.at[j]], rows_v.at[j], sem).wait()
        return c
    lax.fori_loop(0, CH, drain, 0)
    pltpu.sync_copy(rows_v, rows_out.at[wid])


TILE = 256
NT = B // TILE


def _tc_body(dense_ref, xT_ref, wgT_ref, W1T_ref, b1_ref, g1_ref, be1_ref,
             W2T_ref, b2_ref, g2_ref, be2_ref, W3T_ref, bsum_ref, out_ref,
             h1_s, h2_s, yp_s):
    f32 = jnp.float32
    hp = jax.lax.Precision.HIGHEST

    xT = xT_ref[...]
    wgT = wgT_ref[...]
    y_lin32 = wgT[0]
    for f in range(1, F):
        dup = xT[f] == xT[0]
        for f2 in range(1, f):
            dup = dup | (xT[f] == xT[f2])
        y_lin32 = y_lin32 + jnp.where(dup, 0.0, wgT[f])

    ii = lax.broadcasted_iota(jnp.int32, (F * D, D), 0)
    jj = lax.broadcasted_iota(jnp.int32, (F * D, D), 1)
    S = (ii % D == jj).astype(f32)

    W1T = W1T_ref[...]

    def pass1(t, carry):
        s1, q1 = carry
        d = dense_ref[pl.ds(t * TILE, TILE), :]
        h1 = lax.dot(d, W1T, precision=hp) + b1_ref[...]
        h1_s[pl.ds(t * TILE, TILE), :] = h1
        s = lax.dot(d, S, precision=hp)
        ss = lax.dot(d * d, S, precision=hp)
        y_pair = 0.5 * jnp.sum(s * s - ss, axis=1, keepdims=True)
        rsel = (lax.broadcasted_iota(jnp.int32, (TILE, B // 128), 1)
                == 2 * t + lax.broadcasted_iota(jnp.int32, (TILE, B // 128), 0)
                // 128).astype(f32)
        rows128 = lax.dot(rsel, y_lin32, precision=hp)
        lmask = (lax.broadcasted_iota(jnp.int32, (TILE, 128), 1)
                 == lax.broadcasted_iota(jnp.int32, (TILE, 128), 0) % 128)
        y_lin = jnp.sum(jnp.where(lmask, rows128, 0.0), axis=1, keepdims=True)
        yp_s[pl.ds(t * TILE, TILE), :] = y_pair + y_lin + bsum_ref[...]
        return (s1 + jnp.sum(h1, axis=0, keepdims=True),
                q1 + jnp.sum(h1 * h1, axis=0, keepdims=True))

    z1 = jnp.zeros((1, H1), f32)
    s1, q1 = lax.fori_loop(0, NT, pass1, (z1, z1))
    mu1 = s1 / B
    var1 = q1 / B - mu1 * mu1
    sc1 = lax.rsqrt(var1 + EPS) * g1_ref[...]
    sh1 = be1_ref[...] - mu1 * sc1

    W2T = W2T_ref[...]

    def pass2(t, carry):
        s2, q2 = carry
        h1 = h1_s[pl.ds(t * TILE, TILE), :]
        a1 = jnp.maximum(h1 * sc1 + sh1, 0.0)
        h2 = lax.dot(a1, W2T, precision=hp) + b2_ref[...]
        h2_s[pl.ds(t * TILE, TILE), :] = h2
        return (s2 + jnp.sum(h2, axis=0, keepdims=True),
                q2 + jnp.sum(h2 * h2, axis=0, keepdims=True))

    z2 = jnp.zeros((1, H2), f32)
    s2, q2 = lax.fori_loop(0, NT, pass2, (z2, z2))
    mu2 = s2 / B
    var2 = q2 / B - mu2 * mu2
    sc2 = lax.rsqrt(var2 + EPS) * g2_ref[...]
    sh2 = be2_ref[...] - mu2 * sc2

    W3T = W3T_ref[...]

    def pass3(t, c):
        h2 = h2_s[pl.ds(t * TILE, TILE), :]
        a2 = jnp.maximum(h2 * sc2 + sh2, 0.0)
        y_dnn = lax.dot(a2, W3T, precision=hp)
        out_ref[pl.ds(t * TILE, TILE), :] = yp_s[pl.ds(t * TILE, TILE), :] + y_dnn
        return c

    lax.fori_loop(0, NT, pass3, 0)


def kernel(x, emb, w_lin, b_lin, W1, b1, g1, be1, W2, b2, g2, be2, W3, b3):
    flat_idx = (x + jnp.arange(F, dtype=jnp.int32)[None, :] * V)
    idx2d = flat_idx.reshape(NW, CH, 128)
    xfm = x.T.reshape(-1)
    emb_flat = emb.reshape(F * V, D)
    wtab = w_lin.reshape(-1)

    mesh = plsc.VectorSubcoreMesh(core_axis_name="c", subcore_axis_name="s")
    sc = functools.partial(
        pl.kernel,
        mesh=mesh,
        compiler_params=pltpu.CompilerParams(
            needs_layout_passes=False, use_tc_tiling_on_sc=False),
        out_type=(
            jax.ShapeDtypeStruct((NW, CH, 128, D), jnp.float32),
            jax.ShapeDtypeStruct((B * F,), jnp.float32),
        ),
        scratch_types=[
            pltpu.VMEM((CH, 128), jnp.int32),
            pltpu.VMEM((CH, 128, D), jnp.float32),
            pltpu.VMEM((E_PER_W,), jnp.int32),
            pltpu.VMEM((E_PER_W,), jnp.float32),
            pltpu.VMEM((V,), jnp.float32),
            pltpu.SemaphoreType.DMA,
        ],
    )(_sc_gather)
    rows, wg = sc(idx2d, xfm, wtab, emb_flat)

    dense = rows.reshape(B, F * D)
    xT3 = x.T.reshape(F, B // 128, 128)
    wgT3 = wg.reshape(F, B // 128, 128)
    bsum = (b_lin + b3).reshape(1, 1)

    out = pl.pallas_call(
        _tc_body,
        out_shape=jax.ShapeDtypeStruct((B, 1), jnp.float32),
        scratch_shapes=[
            pltpu.VMEM((B, H1), jnp.float32),
            pltpu.VMEM((B, H2), jnp.float32),
            pltpu.VMEM((B, 1), jnp.float32),
        ],
    )(dense, xT3, wgT3,
      W1.T, b1.reshape(1, H1), g1.reshape(1, H1), be1.reshape(1, H1),
      W2.T, b2.reshape(1, H2), g2.reshape(1, H2), be2.reshape(1, H2),
      W3.T, bsum)
    return out.reshape(B)


# speedup vs baseline: 36.7096x; 1.2569x over previous
import functools

import jax
import jax.numpy as jnp
from jax import lax
from jax.experimental import pallas as pl
from jax.experimental.pallas import tpu as pltpu
from jax.experimental.pallas import tpu_sc as plsc

B = 4096
F = 26
V = 1000
D = 16
H1, H2 = 64, 32
EPS = 1e-5

NC = 2
NS = 16
NW = NC * NS
E_PER_W = (B * F) // NW
CH = E_PER_W // 128
LANE_STEPS = E_PER_W // 16


def _sc_gather(idx2d_hbm, xfm_hbm, wtab_hbm, emb_hbm,
               rows_out, wg_out,
               idx_v, rows_v, xv, wgv, wtab, sem):
    wid = lax.axis_index("s") * NC + lax.axis_index("c")
    base = wid * E_PER_W

    pltpu.sync_copy(idx2d_hbm.at[wid], idx_v)
    pltpu.sync_copy(xfm_hbm.at[pl.ds(base, E_PER_W)], xv)
    pltpu.sync_copy(wtab_hbm.at[pl.ds(0, V)], wtab)

    def fire(j, c):
        pltpu.async_copy(emb_hbm.at[idx_v.at[j]], rows_v.at[j], sem)
        return c
    lax.fori_loop(0, CH, fire, 0)

    def wbody(i, c):
        idx = xv[pl.ds(i * 16, 16)]
        wgv[pl.ds(i * 16, 16)] = plsc.load_gather(wtab, [idx])
        return c
    lax.fori_loop(0, LANE_STEPS, wbody, 0)
    pltpu.sync_copy(wgv, wg_out.at[pl.ds(base, E_PER_W)])

    def drain(j, c):
        pltpu.make_async_copy(emb_hbm.at[idx_v.at[j]], rows_v.at[j], sem).wait()
        return c
    lax.fori_loop(0, CH, drain, 0)
    pltpu.sync_copy(rows_v, rows_out.at[wid])


TILE = 256
NT = B // TILE


def _tc_body(dense_ref, xT_ref, wgT_ref, W1T_ref, b1_ref, g1_ref, be1_ref,
             W2T_ref, b2_ref, g2_ref, be2_ref, W3T_ref, bsum_ref, out_ref,
             h1_s, h2_s, yp_s):
    f32 = jnp.float32

    xT = xT_ref[...]
    wgT = wgT_ref[...]
    y_lin32 = wgT[0]
    for f in range(1, F):
        dup = xT[f] == xT[0]
        for f2 in range(1, f):
            dup = dup | (xT[f] == xT[f2])
        y_lin32 = y_lin32 + jnp.where(dup, 0.0, wgT[f])

    ii = lax.broadcasted_iota(jnp.int32, (F * D, D), 0)
    jj = lax.broadcasted_iota(jnp.int32, (F * D, D), 1)
    S = (ii % D == jj).astype(f32)

    W1T = W1T_ref[...]

    def pass1(t, carry):
        s1, q1 = carry
        d = dense_ref[pl.ds(t * TILE, TILE), :]
        h1 = lax.dot(d, W1T) + b1_ref[...]
        h1_s[pl.ds(t * TILE, TILE), :] = h1
        s = lax.dot(d, S)
        ss = lax.dot(d * d, S)
        y_pair = 0.5 * jnp.sum(s * s - ss, axis=1, keepdims=True)
        rsel = (lax.broadcasted_iota(jnp.int32, (TILE, B // 128), 1)
                == 2 * t + lax.broadcasted_iota(jnp.int32, (TILE, B // 128), 0)
                // 128).astype(f32)
        rows128 = lax.dot(rsel, y_lin32)
        lmask = (lax.broadcasted_iota(jnp.int32, (TILE, 128), 1)
                 == lax.broadcasted_iota(jnp.int32, (TILE, 128), 0) % 128)
        y_lin = jnp.sum(jnp.where(lmask, rows128, 0.0), axis=1, keepdims=True)
        yp_s[pl.ds(t * TILE, TILE), :] = y_pair + y_lin + bsum_ref[...]
        return (s1 + jnp.sum(h1, axis=0, keepdims=True),
                q1 + jnp.sum(h1 * h1, axis=0, keepdims=True))

    z1 = jnp.zeros((1, H1), f32)
    s1, q1 = lax.fori_loop(0, NT, pass1, (z1, z1))
    mu1 = s1 / B
    var1 = q1 / B - mu1 * mu1
    sc1 = lax.rsqrt(var1 + EPS) * g1_ref[...]
    sh1 = be1_ref[...] - mu1 * sc1

    W2T = W2T_ref[...]

    def pass2(t, carry):
        s2, q2 = carry
        h1 = h1_s[pl.ds(t * TILE, TILE), :]
        a1 = jnp.maximum(h1 * sc1 + sh1, 0.0)
        h2 = lax.dot(a1, W2T) + b2_ref[...]
        h2_s[pl.ds(t * TILE, TILE), :] = h2
        return (s2 + jnp.sum(h2, axis=0, keepdims=True),
                q2 + jnp.sum(h2 * h2, axis=0, keepdims=True))

    z2 = jnp.zeros((1, H2), f32)
    s2, q2 = lax.fori_loop(0, NT, pass2, (z2, z2))
    mu2 = s2 / B
    var2 = q2 / B - mu2 * mu2
    sc2 = lax.rsqrt(var2 + EPS) * g2_ref[...]
    sh2 = be2_ref[...] - mu2 * sc2

    W3T = W3T_ref[...]

    def pass3(t, c):
        h2 = h2_s[pl.ds(t * TILE, TILE), :]
        a2 = jnp.maximum(h2 * sc2 + sh2, 0.0)
        y_dnn = lax.dot(a2, W3T)
        out_ref[pl.ds(t * TILE, TILE), :] = yp_s[pl.ds(t * TILE, TILE), :] + y_dnn
        return c

    lax.fori_loop(0, NT, pass3, 0)


def kernel(x, emb, w_lin, b_lin, W1, b1, g1, be1, W2, b2, g2, be2, W3, b3):
    flat_idx = (x + jnp.arange(F, dtype=jnp.int32)[None, :] * V)
    idx2d = flat_idx.reshape(NW, CH, 128)
    xfm = x.T.reshape(-1)
    emb_flat = emb.reshape(F * V, D)
    wtab = w_lin.reshape(-1)

    mesh = plsc.VectorSubcoreMesh(core_axis_name="c", subcore_axis_name="s")
    sc = functools.partial(
        pl.kernel,
        mesh=mesh,
        compiler_params=pltpu.CompilerParams(
            needs_layout_passes=False, use_tc_tiling_on_sc=False),
        out_type=(
            jax.ShapeDtypeStruct((NW, CH, 128, D), jnp.float32),
            jax.ShapeDtypeStruct((B * F,), jnp.float32),
        ),
        scratch_types=[
            pltpu.VMEM((CH, 128), jnp.int32),
            pltpu.VMEM((CH, 128, D), jnp.float32),
            pltpu.VMEM((E_PER_W,), jnp.int32),
            pltpu.VMEM((E_PER_W,), jnp.float32),
            pltpu.VMEM((V,), jnp.float32),
            pltpu.SemaphoreType.DMA,
        ],
    )(_sc_gather)
    rows, wg = sc(idx2d, xfm, wtab, emb_flat)

    dense = rows.reshape(B, F * D)
    xT3 = x.T.reshape(F, B // 128, 128)
    wgT3 = wg.reshape(F, B // 128, 128)
    bsum = (b_lin + b3).reshape(1, 1)

    out = pl.pallas_call(
        _tc_body,
        out_shape=jax.ShapeDtypeStruct((B, 1), jnp.float32),
        scratch_shapes=[
            pltpu.VMEM((B, H1), jnp.float32),
            pltpu.VMEM((B, H2), jnp.float32),
            pltpu.VMEM((B, 1), jnp.float32),
        ],
    )(dense, xT3, wgT3,
      W1.T, b1.reshape(1, H1), g1.reshape(1, H1), be1.reshape(1, H1),
      W2.T, b2.reshape(1, H2), g2.reshape(1, H2), be2.reshape(1, H2),
      W3.T, bsum)
    return out.reshape(B)
